# baseline (speedup 1.0000x reference)
import jax
import jax.numpy as jnp
from jax.experimental import pallas as pl
from jax.experimental.pallas import tpu as pltpu

_TILE = 64


def _route_body(eid_ref, slot_ref, tos_ref, starts_ref, nblocks_ref):
    nk = eid_ref.shape[0]
    num_e = starts_ref.shape[1]
    npad = tos_ref.shape[1]

    eid = eid_ref[...]
    lanes = jax.lax.broadcasted_iota(jnp.int32, (nk, num_e), 1)
    oh = eid == lanes
    oh_bf = oh.astype(jnp.bfloat16)
    oh_f = oh.astype(jnp.float32)

    row = jax.lax.broadcasted_iota(jnp.int32, (nk, nk), 0)
    col = jax.lax.broadcasted_iota(jnp.int32, (nk, nk), 1)
    lower = (col < row).astype(jnp.bfloat16)
    before = jnp.dot(lower, oh_bf, preferred_element_type=jnp.float32)
    rank = jnp.sum(before * oh_f, axis=1, keepdims=True)

    counts = jnp.sum(oh_f, axis=0, keepdims=True).astype(jnp.int32)
    aligned = ((counts + 7) // 8) * 8
    erow = jax.lax.broadcasted_iota(jnp.int32, (num_e, num_e), 0)
    ecol = jax.lax.broadcasted_iota(jnp.int32, (num_e, num_e), 1)
    tri = (erow < ecol).astype(jnp.bfloat16)
    starts_f = jnp.dot(aligned.astype(jnp.bfloat16), tri,
                       preferred_element_type=jnp.float32)
    start_of_tok = jnp.sum(starts_f * oh_f, axis=1, keepdims=True)
    slot = (start_of_tok + rank).astype(jnp.int32)

    slot_ref[...] = slot
    starts_ref[...] = starts_f.astype(jnp.int32)
    nblocks_ref[...] = (counts + (_TILE - 1)) // _TILE

    jlane = jax.lax.broadcasted_iota(jnp.int32, (nk, npad), 1)
    sel = (slot == jlane).astype(jnp.bfloat16)
    tok = jax.lax.broadcasted_iota(jnp.int32, (1, nk), 1)
    q_row = (tok // 16).astype(jnp.bfloat16)
    r_row = (tok % 16).astype(jnp.bfloat16)
    hi = jnp.dot(q_row, sel, preferred_element_type=jnp.float32)
    lo = jnp.dot(r_row, sel, preferred_element_type=jnp.float32)
    tos_ref[...] = (16.0 * hi + lo).astype(jnp.int32)


def _moe_body(starts_ref, nblocks_ref, xs_ref, xw_ref, wg_ref, wu_ref,
              wd_ref, out_ref):
    e = pl.program_id(0)
    start = starts_ref[e]
    nb = nblocks_ref[e]
    wg = wg_ref[0].astype(jnp.bfloat16)
    wu = wu_ref[0].astype(jnp.bfloat16)
    wd = wd_ref[0].astype(jnp.bfloat16)

    def tile(k, carry):
        offs = pl.multiple_of(start + k * _TILE, 8)
        x = xs_ref[pl.ds(offs, _TILE), :].astype(jnp.bfloat16)
        xw = xw_ref[pl.ds(offs, _TILE), :].astype(jnp.bfloat16)
        g = jnp.dot(x, wg, preferred_element_type=jnp.float32)
        u = jnp.dot(xw, wu, preferred_element_type=jnp.float32)
        a = ((g * jax.nn.sigmoid(g)) * u).astype(jnp.bfloat16)
        out_ref[pl.ds(offs, _TILE), :] = jnp.dot(
            a, wd, preferred_element_type=jnp.float32)
        return carry

    jax.lax.fori_loop(0, nb, tile, 0)


def kernel(hidden_states, top_k_indices, top_k_weights, Wg, Wu, Wd):
    N, D = hidden_states.shape
    E, _, H = Wg.shape
    K = top_k_indices.shape[1]
    NK = N * K

    npad = NK + 8 * E + 4 * _TILE
    npad = ((npad + 255) // 256) * 256

    eid = top_k_indices.reshape(NK, 1).astype(jnp.int32)
    wts = top_k_weights.reshape(NK).astype(jnp.float32)

    slot, tos, starts, nblocks = pl.pallas_call(
        _route_body,
        out_shape=(
            jax.ShapeDtypeStruct((NK, 1), jnp.int32),
            jax.ShapeDtypeStruct((1, npad), jnp.int32),
            jax.ShapeDtypeStruct((1, E), jnp.int32),
            jax.ShapeDtypeStruct((1, E), jnp.int32),
        ),
    )(eid)

    if K > 1:
        hs = hidden_states[
            jnp.repeat(jnp.arange(N, dtype=jnp.int32), K)]
    else:
        hs = hidden_states
    hw = hs * wts[:, None]
    tos_vec = tos.reshape(npad)
    xs = hs[tos_vec]
    xw = hw[tos_vec]

    ys = pl.pallas_call(
        _moe_body,
        grid_spec=pltpu.PrefetchScalarGridSpec(
            num_scalar_prefetch=2,
            grid=(E,),
            in_specs=[
                pl.BlockSpec((npad, D), lambda e, s, nb: (0, 0)),
                pl.BlockSpec((npad, D), lambda e, s, nb: (0, 0)),
                pl.BlockSpec((1, D, H), lambda e, s, nb: (e, 0, 0)),
                pl.BlockSpec((1, D, H), lambda e, s, nb: (e, 0, 0)),
                pl.BlockSpec((1, H, D), lambda e, s, nb: (e, 0, 0)),
            ],
            out_specs=pl.BlockSpec((npad, D), lambda e, s, nb: (0, 0)),
        ),
        out_shape=jax.ShapeDtypeStruct((npad, D), jnp.float32),
        compiler_params=pltpu.CompilerParams(
            dimension_semantics=("arbitrary",)),
    )(starts.reshape(E), nblocks.reshape(E), xs, xw, Wg, Wu, Wd)

    slot2 = slot.reshape(N, K)
    out = ys[slot2[:, 0]]
    for k in range(1, K):
        out = out + ys[slot2[:, k]]
    return out


# speedup vs baseline: 1.3689x; 1.3689x over previous
import functools

import jax
import jax.numpy as jnp
from jax.experimental import pallas as pl
from jax.experimental.pallas import tpu as pltpu
from jax.experimental.pallas import tpu_sc as plsc

_TILE = 64


def _route_body(eid_ref, slot_ref, starts_ref, nblocks_ref):
    nk = eid_ref.shape[0]
    num_e = starts_ref.shape[1]

    eid = eid_ref[...]
    lanes = jax.lax.broadcasted_iota(jnp.int32, (nk, num_e), 1)
    oh = eid == lanes
    oh_bf = oh.astype(jnp.bfloat16)
    oh_f = oh.astype(jnp.float32)

    row = jax.lax.broadcasted_iota(jnp.int32, (nk, nk), 0)
    col = jax.lax.broadcasted_iota(jnp.int32, (nk, nk), 1)
    lower = (col < row).astype(jnp.bfloat16)
    before = jnp.dot(lower, oh_bf, preferred_element_type=jnp.float32)
    rank = jnp.sum(before * oh_f, axis=1, keepdims=True)

    counts = jnp.sum(oh_f, axis=0, keepdims=True).astype(jnp.int32)
    aligned = ((counts + 7) // 8) * 8
    erow = jax.lax.broadcasted_iota(jnp.int32, (num_e, num_e), 0)
    ecol = jax.lax.broadcasted_iota(jnp.int32, (num_e, num_e), 1)
    tri = (erow < ecol).astype(jnp.bfloat16)
    starts_f = jnp.dot(aligned.astype(jnp.bfloat16), tri,
                       preferred_element_type=jnp.float32)
    start_of_tok = jnp.sum(starts_f * oh_f, axis=1, keepdims=True)

    slot_ref[...] = (start_of_tok + rank).astype(jnp.int32)
    starts_ref[...] = starts_f.astype(jnp.int32)
    nblocks_ref[...] = (counts + (_TILE - 1)) // _TILE


def _dispatch_sc(hs, hw, slot, npad):
    n, d = hs.shape
    info = plsc.get_sparse_core_info()
    nw = info.num_cores * info.num_subcores
    chunk = n // nw
    idx2d = slot.reshape(nw, chunk)
    mesh = plsc.VectorSubcoreMesh(core_axis_name="c", subcore_axis_name="s")

    @functools.partial(
        pl.kernel,
        mesh=mesh,
        out_type=(
            jax.ShapeDtypeStruct((npad, d), jnp.float32),
            jax.ShapeDtypeStruct((npad, d), jnp.float32),
        ),
        scratch_types=[
            pltpu.VMEM((chunk,), jnp.int32),
            pltpu.VMEM((chunk, d), jnp.float32),
            pltpu.SemaphoreType.DMA,
        ],
    )
    def scat(hs_hbm, hw_hbm, idx_hbm, xs_hbm, xw_hbm, idx_v, rows_v, sem):
        wid = (jax.lax.axis_index("s") * info.num_cores
               + jax.lax.axis_index("c"))
        base = wid * chunk
        pltpu.sync_copy(idx_hbm.at[wid], idx_v)
        pltpu.sync_copy(hs_hbm.at[pl.ds(base, chunk)], rows_v)
        pltpu.async_copy(rows_v, xs_hbm.at[idx_v], sem).wait()
        pltpu.sync_copy(hw_hbm.at[pl.ds(base, chunk)], rows_v)
        pltpu.async_copy(rows_v, xw_hbm.at[idx_v], sem).wait()

    return scat(hs, hw, idx2d)


def _moe_body(starts_ref, nblocks_ref, xs_ref, xw_ref, wg_ref, wu_ref,
              wd_ref, out_ref):
    e = pl.program_id(0)
    start = starts_ref[e]
    nb = nblocks_ref[e]
    wg = wg_ref[0].astype(jnp.bfloat16)
    wu = wu_ref[0].astype(jnp.bfloat16)
    wd = wd_ref[0].astype(jnp.bfloat16)

    def tile(k, carry):
        offs = pl.multiple_of(start + k * _TILE, 8)
        x = xs_ref[pl.ds(offs, _TILE), :].astype(jnp.bfloat16)
        xw = xw_ref[pl.ds(offs, _TILE), :].astype(jnp.bfloat16)
        g = jnp.dot(x, wg, preferred_element_type=jnp.float32)
        u = jnp.dot(xw, wu, preferred_element_type=jnp.float32)
        a = ((g * jax.nn.sigmoid(g)) * u).astype(jnp.bfloat16)
        out_ref[pl.ds(offs, _TILE), :] = jnp.dot(
            a, wd, preferred_element_type=jnp.float32)
        return carry

    jax.lax.fori_loop(0, nb, tile, 0)


def kernel(hidden_states, top_k_indices, top_k_weights, Wg, Wu, Wd):
    N, D = hidden_states.shape
    E, _, H = Wg.shape
    K = top_k_indices.shape[1]
    NK = N * K

    npad = NK + 8 * E + 4 * _TILE
    npad = ((npad + 255) // 256) * 256

    eid = top_k_indices.reshape(NK, 1).astype(jnp.int32)
    wts = top_k_weights.reshape(NK).astype(jnp.float32)

    slot, starts, nblocks = pl.pallas_call(
        _route_body,
        out_shape=(
            jax.ShapeDtypeStruct((NK, 1), jnp.int32),
            jax.ShapeDtypeStruct((1, E), jnp.int32),
            jax.ShapeDtypeStruct((1, E), jnp.int32),
        ),
    )(eid)
    slot = slot.reshape(NK)

    if K > 1:
        hs = hidden_states[
            jnp.repeat(jnp.arange(N, dtype=jnp.int32), K)]
    else:
        hs = hidden_states
    hw = hs * wts[:, None]
    xs, xw = _dispatch_sc(hs, hw, slot, npad)

    ys = pl.pallas_call(
        _moe_body,
        grid_spec=pltpu.PrefetchScalarGridSpec(
            num_scalar_prefetch=2,
            grid=(E,),
            in_specs=[
                pl.BlockSpec((npad, D), lambda e, s, nb: (0, 0)),
                pl.BlockSpec((npad, D), lambda e, s, nb: (0, 0)),
                pl.BlockSpec((1, D, H), lambda e, s, nb: (e, 0, 0)),
                pl.BlockSpec((1, D, H), lambda e, s, nb: (e, 0, 0)),
                pl.BlockSpec((1, H, D), lambda e, s, nb: (e, 0, 0)),
            ],
            out_specs=pl.BlockSpec((npad, D), lambda e, s, nb: (0, 0)),
        ),
        out_shape=jax.ShapeDtypeStruct((npad, D), jnp.float32),
        compiler_params=pltpu.CompilerParams(
            dimension_semantics=("arbitrary",)),
    )(starts.reshape(E), nblocks.reshape(E), xs, xw, Wg, Wu, Wd)

    slot2 = slot.reshape(N, K)
    out = ys[slot2[:, 0]]
    for k in range(1, K):
        out = out + ys[slot2[:, k]]
    return out


# speedup vs baseline: 1.3985x; 1.0216x over previous
import functools

import jax
import jax.numpy as jnp
from jax.experimental import pallas as pl
from jax.experimental.pallas import tpu as pltpu
from jax.experimental.pallas import tpu_sc as plsc

_TILE = 64


def _route_body(eid_ref, slot_ref, starts_ref, nblocks_ref):
    nk = eid_ref.shape[0]
    num_e = starts_ref.shape[1]

    eid = eid_ref[...]
    lanes = jax.lax.broadcasted_iota(jnp.int32, (nk, num_e), 1)
    oh = eid == lanes
    oh_bf = oh.astype(jnp.bfloat16)
    oh_f = oh.astype(jnp.float32)

    row = jax.lax.broadcasted_iota(jnp.int32, (nk, nk), 0)
    col = jax.lax.broadcasted_iota(jnp.int32, (nk, nk), 1)
    lower = (col < row).astype(jnp.bfloat16)
    before = jnp.dot(lower, oh_bf, preferred_element_type=jnp.float32)
    rank = jnp.sum(before * oh_f, axis=1, keepdims=True)

    counts = jnp.sum(oh_f, axis=0, keepdims=True).astype(jnp.int32)
    aligned = ((counts + 7) // 8) * 8
    erow = jax.lax.broadcasted_iota(jnp.int32, (num_e, num_e), 0)
    ecol = jax.lax.broadcasted_iota(jnp.int32, (num_e, num_e), 1)
    tri = (erow < ecol).astype(jnp.bfloat16)
    starts_f = jnp.dot(aligned.astype(jnp.bfloat16), tri,
                       preferred_element_type=jnp.float32)
    start_of_tok = jnp.sum(starts_f * oh_f, axis=1, keepdims=True)

    slot_ref[...] = (start_of_tok + rank).astype(jnp.int32)
    starts_ref[...] = starts_f.astype(jnp.int32)
    nblocks_ref[...] = (counts + (_TILE - 1)) // _TILE


def _dispatch_sc(hs, w128, slot, npad):
    n, d = hs.shape
    dw = w128.shape[1]
    info = plsc.get_sparse_core_info()
    nw = info.num_cores * info.num_subcores
    chunk = n // nw
    idx2d = slot.reshape(nw, chunk)
    mesh = plsc.VectorSubcoreMesh(core_axis_name="c", subcore_axis_name="s")

    @functools.partial(
        pl.kernel,
        mesh=mesh,
        out_type=(
            jax.ShapeDtypeStruct((npad, d), jnp.float32),
            jax.ShapeDtypeStruct((npad, dw), jnp.float32),
        ),
        scratch_types=[
            pltpu.VMEM((chunk,), jnp.int32),
            pltpu.VMEM((chunk, d), jnp.float32),
            pltpu.VMEM((chunk, dw), jnp.float32),
            pltpu.SemaphoreType.DMA,
        ],
    )
    def scat(hs_hbm, w_hbm, idx_hbm, xs_hbm, ws_hbm, idx_v, rows_v, w_v,
             sem):
        wid = (jax.lax.axis_index("s") * info.num_cores
               + jax.lax.axis_index("c"))
        base = wid * chunk
        pltpu.sync_copy(idx_hbm.at[wid], idx_v)
        pltpu.sync_copy(hs_hbm.at[pl.ds(base, chunk)], rows_v)
        pltpu.async_copy(rows_v, xs_hbm.at[idx_v], sem).wait()
        pltpu.sync_copy(w_hbm.at[pl.ds(base, chunk)], w_v)
        pltpu.async_copy(w_v, ws_hbm.at[idx_v], sem).wait()

    return scat(hs, w128, idx2d)


def _moe_body(starts_ref, nblocks_ref, xs_ref, ws_ref, wg_ref, wu_ref,
              wd_ref, out_ref):
    e = pl.program_id(0)
    start = starts_ref[e]
    nb = nblocks_ref[e]
    wg = wg_ref[0].astype(jnp.bfloat16)
    wu = wu_ref[0].astype(jnp.bfloat16)
    wd = wd_ref[0].astype(jnp.bfloat16)

    def tile(k, carry):
        offs = pl.multiple_of(start + k * _TILE, 8)
        x = xs_ref[pl.ds(offs, _TILE), :].astype(jnp.bfloat16)
        g = jnp.dot(x, wg, preferred_element_type=jnp.float32)
        u = jnp.dot(x, wu, preferred_element_type=jnp.float32)
        a = ((g * jax.nn.sigmoid(g)) * u).astype(jnp.bfloat16)
        o = jnp.dot(a, wd, preferred_element_type=jnp.float32)
        w = ws_ref[pl.ds(offs, _TILE), 0:1]
        out_ref[pl.ds(offs, _TILE), :] = o * w
        return carry

    jax.lax.fori_loop(0, nb, tile, 0)


def kernel(hidden_states, top_k_indices, top_k_weights, Wg, Wu, Wd):
    N, D = hidden_states.shape
    E, _, H = Wg.shape
    K = top_k_indices.shape[1]
    NK = N * K

    npad = NK + 8 * E + 4 * _TILE
    npad = ((npad + 255) // 256) * 256

    eid = top_k_indices.reshape(NK, 1).astype(jnp.int32)
    wts = top_k_weights.reshape(NK).astype(jnp.float32)

    slot, starts, nblocks = pl.pallas_call(
        _route_body,
        out_shape=(
            jax.ShapeDtypeStruct((NK, 1), jnp.int32),
            jax.ShapeDtypeStruct((1, E), jnp.int32),
            jax.ShapeDtypeStruct((1, E), jnp.int32),
        ),
    )(eid)
    slot = slot.reshape(NK)

    if K > 1:
        hs = hidden_states[
            jnp.repeat(jnp.arange(N, dtype=jnp.int32), K)]
    else:
        hs = hidden_states
    w128 = jnp.broadcast_to(wts[:, None], (NK, 128))
    xs, ws = _dispatch_sc(hs, w128, slot, npad)

    ys = pl.pallas_call(
        _moe_body,
        grid_spec=pltpu.PrefetchScalarGridSpec(
            num_scalar_prefetch=2,
            grid=(E,),
            in_specs=[
                pl.BlockSpec((npad, D), lambda e, s, nb: (0, 0)),
                pl.BlockSpec((npad, 128), lambda e, s, nb: (0, 0)),
                pl.BlockSpec((1, D, H), lambda e, s, nb: (e, 0, 0)),
                pl.BlockSpec((1, D, H), lambda e, s, nb: (e, 0, 0)),
                pl.BlockSpec((1, H, D), lambda e, s, nb: (e, 0, 0)),
            ],
            out_specs=pl.BlockSpec((npad, D), lambda e, s, nb: (0, 0)),
        ),
        out_shape=jax.ShapeDtypeStruct((npad, D), jnp.float32),
        compiler_params=pltpu.CompilerParams(
            dimension_semantics=("arbitrary",)),
    )(starts.reshape(E), nblocks.reshape(E), xs, ws, Wg, Wu, Wd)

    slot2 = slot.reshape(N, K)
    out = ys[slot2[:, 0]]
    for k in range(1, K):
        out = out + ys[slot2[:, k]]
    return out


# speedup vs baseline: 1.4234x; 1.0178x over previous
import functools

import jax
import jax.numpy as jnp
from jax.experimental import pallas as pl
from jax.experimental.pallas import tpu as pltpu
from jax.experimental.pallas import tpu_sc as plsc

_TILE = 64


def _route_body(eid_ref, wts_ref, slot_ref, w128_ref, starts_ref,
                nblocks_ref):
    nk = eid_ref.shape[0]
    num_e = starts_ref.shape[1]
    w128_ref[...] = jnp.broadcast_to(wts_ref[...], (nk, 128))

    eid = eid_ref[...]
    lanes = jax.lax.broadcasted_iota(jnp.int32, (nk, num_e), 1)
    oh = eid == lanes
    oh_bf = oh.astype(jnp.bfloat16)
    oh_f = oh.astype(jnp.float32)

    row = jax.lax.broadcasted_iota(jnp.int32, (nk, nk), 0)
    col = jax.lax.broadcasted_iota(jnp.int32, (nk, nk), 1)
    lower = (col < row).astype(jnp.bfloat16)
    before = jnp.dot(lower, oh_bf, preferred_element_type=jnp.float32)
    rank = jnp.sum(before * oh_f, axis=1, keepdims=True)

    counts = jnp.sum(oh_f, axis=0, keepdims=True).astype(jnp.int32)
    aligned = ((counts + 7) // 8) * 8
    erow = jax.lax.broadcasted_iota(jnp.int32, (num_e, num_e), 0)
    ecol = jax.lax.broadcasted_iota(jnp.int32, (num_e, num_e), 1)
    tri = (erow < ecol).astype(jnp.bfloat16)
    starts_f = jnp.dot(aligned.astype(jnp.bfloat16), tri,
                       preferred_element_type=jnp.float32)
    start_of_tok = jnp.sum(starts_f * oh_f, axis=1, keepdims=True)

    slot_ref[...] = (start_of_tok + rank).astype(jnp.int32)
    starts_ref[...] = starts_f.astype(jnp.int32)
    nblocks_ref[...] = (counts + (_TILE - 1)) // _TILE


def _dispatch_sc(hs, w128, slot, npad):
    n, d = hs.shape
    dw = w128.shape[1]
    info = plsc.get_sparse_core_info()
    nw = info.num_cores * info.num_subcores
    chunk = n // nw
    idx2d = slot.reshape(nw, chunk)
    mesh = plsc.VectorSubcoreMesh(core_axis_name="c", subcore_axis_name="s")

    @functools.partial(
        pl.kernel,
        mesh=mesh,
        out_type=(
            jax.ShapeDtypeStruct((npad, d), jnp.float32),
            jax.ShapeDtypeStruct((npad, dw), jnp.float32),
        ),
        scratch_types=[
            pltpu.VMEM((chunk,), jnp.int32),
            pltpu.VMEM((chunk, d), jnp.float32),
            pltpu.VMEM((chunk, dw), jnp.float32),
            pltpu.SemaphoreType.DMA,
        ],
    )
    def scat(hs_hbm, w_hbm, idx_hbm, xs_hbm, ws_hbm, idx_v, rows_v, w_v,
             sem):
        wid = (jax.lax.axis_index("s") * info.num_cores
               + jax.lax.axis_index("c"))
        base = wid * chunk
        pltpu.sync_copy(idx_hbm.at[wid], idx_v)
        pltpu.sync_copy(hs_hbm.at[pl.ds(base, chunk)], rows_v)
        pltpu.async_copy(rows_v, xs_hbm.at[idx_v], sem).wait()
        pltpu.sync_copy(w_hbm.at[pl.ds(base, chunk)], w_v)
        pltpu.async_copy(w_v, ws_hbm.at[idx_v], sem).wait()

    return scat(hs, w128, idx2d)


def _unsort_sc(ys, slot, n):
    npad, d = ys.shape
    info = plsc.get_sparse_core_info()
    nw = info.num_cores * info.num_subcores
    chunk = n // nw
    idx2d = slot.reshape(nw, chunk)
    mesh = plsc.VectorSubcoreMesh(core_axis_name="c", subcore_axis_name="s")

    @functools.partial(
        pl.kernel,
        mesh=mesh,
        out_type=jax.ShapeDtypeStruct((n, d), jnp.float32),
        scratch_types=[
            pltpu.VMEM((chunk,), jnp.int32),
            pltpu.VMEM((chunk, d), jnp.float32),
            pltpu.SemaphoreType.DMA,
        ],
    )
    def gath(ys_hbm, idx_hbm, out_hbm, idx_v, rows_v, sem):
        wid = (jax.lax.axis_index("s") * info.num_cores
               + jax.lax.axis_index("c"))
        base = wid * chunk
        pltpu.sync_copy(idx_hbm.at[wid], idx_v)
        pltpu.async_copy(ys_hbm.at[idx_v], rows_v, sem).wait()
        pltpu.sync_copy(rows_v, out_hbm.at[pl.ds(base, chunk)])

    return gath(ys, idx2d)


def _moe_body(starts_ref, nblocks_ref, xs_ref, ws_ref, wg_ref, wu_ref,
              wd_ref, out_ref):
    e = pl.program_id(0)
    start = starts_ref[e]
    nb = nblocks_ref[e]
    wg = wg_ref[0].astype(jnp.bfloat16)
    wu = wu_ref[0].astype(jnp.bfloat16)
    wd = wd_ref[0].astype(jnp.bfloat16)

    def tile(k, carry):
        offs = pl.multiple_of(start + k * _TILE, 8)
        x = xs_ref[pl.ds(offs, _TILE), :].astype(jnp.bfloat16)
        g = jnp.dot(x, wg, preferred_element_type=jnp.float32)
        u = jnp.dot(x, wu, preferred_element_type=jnp.float32)
        a = ((g * jax.nn.sigmoid(g)) * u).astype(jnp.bfloat16)
        o = jnp.dot(a, wd, preferred_element_type=jnp.float32)
        w = ws_ref[pl.ds(offs, _TILE), 0:1]
        out_ref[pl.ds(offs, _TILE), :] = o * w
        return carry

    jax.lax.fori_loop(0, nb, tile, 0)


def kernel(hidden_states, top_k_indices, top_k_weights, Wg, Wu, Wd):
    N, D = hidden_states.shape
    E, _, H = Wg.shape
    K = top_k_indices.shape[1]
    NK = N * K

    npad = NK + 8 * E + 4 * _TILE
    npad = ((npad + 255) // 256) * 256

    eid = top_k_indices.reshape(NK, 1).astype(jnp.int32)
    wts = top_k_weights.reshape(NK, 1).astype(jnp.float32)

    slot, w128, starts, nblocks = pl.pallas_call(
        _route_body,
        out_shape=(
            jax.ShapeDtypeStruct((NK, 1), jnp.int32),
            jax.ShapeDtypeStruct((NK, 128), jnp.float32),
            jax.ShapeDtypeStruct((1, E), jnp.int32),
            jax.ShapeDtypeStruct((1, E), jnp.int32),
        ),
    )(eid, wts)
    slot = slot.reshape(NK)

    if K > 1:
        hs = hidden_states[
            jnp.repeat(jnp.arange(N, dtype=jnp.int32), K)]
    else:
        hs = hidden_states
    xs, ws = _dispatch_sc(hs, w128, slot, npad)

    ys = pl.pallas_call(
        _moe_body,
        grid_spec=pltpu.PrefetchScalarGridSpec(
            num_scalar_prefetch=2,
            grid=(E,),
            in_specs=[
                pl.BlockSpec((npad, D), lambda e, s, nb: (0, 0)),
                pl.BlockSpec((npad, 128), lambda e, s, nb: (0, 0)),
                pl.BlockSpec((1, D, H), lambda e, s, nb: (e, 0, 0)),
                pl.BlockSpec((1, D, H), lambda e, s, nb: (e, 0, 0)),
                pl.BlockSpec((1, H, D), lambda e, s, nb: (e, 0, 0)),
            ],
            out_specs=pl.BlockSpec((npad, D), lambda e, s, nb: (0, 0)),
        ),
        out_shape=jax.ShapeDtypeStruct((npad, D), jnp.float32),
        compiler_params=pltpu.CompilerParams(
            dimension_semantics=("arbitrary",)),
    )(starts.reshape(E), nblocks.reshape(E), xs, ws, Wg, Wu, Wd)

    if K == 1:
        return _unsort_sc(ys, slot, N)
    slot2 = slot.reshape(N, K)
    out = ys[slot2[:, 0]]
    for k in range(1, K):
        out = out + ys[slot2[:, k]]
    return out


# speedup vs baseline: 1.4244x; 1.0007x over previous
import functools

import jax
import jax.numpy as jnp
from jax.experimental import pallas as pl
from jax.experimental.pallas import tpu as pltpu
from jax.experimental.pallas import tpu_sc as plsc

_TILE = 64


def _route_body(eid_ref, wts_ref, slot_ref, w128_ref, starts_ref,
                nblocks_ref):
    nk = eid_ref.shape[0]
    num_e = starts_ref.shape[1]
    w128_ref[...] = jnp.broadcast_to(wts_ref[...], (nk, 128))

    eid = eid_ref[...]
    lanes = jax.lax.broadcasted_iota(jnp.int32, (nk, num_e), 1)
    oh = eid == lanes
    oh_bf = oh.astype(jnp.bfloat16)
    oh_f = oh.astype(jnp.float32)

    row = jax.lax.broadcasted_iota(jnp.int32, (nk, nk), 0)
    col = jax.lax.broadcasted_iota(jnp.int32, (nk, nk), 1)
    lower = (col < row).astype(jnp.bfloat16)
    before = jnp.dot(lower, oh_bf, preferred_element_type=jnp.float32)
    rank = jnp.sum(before * oh_f, axis=1, keepdims=True)

    counts = jnp.sum(oh_f, axis=0, keepdims=True).astype(jnp.int32)
    aligned = ((counts + 7) // 8) * 8
    erow = jax.lax.broadcasted_iota(jnp.int32, (num_e, num_e), 0)
    ecol = jax.lax.broadcasted_iota(jnp.int32, (num_e, num_e), 1)
    tri = (erow < ecol).astype(jnp.bfloat16)
    starts_f = jnp.dot(aligned.astype(jnp.bfloat16), tri,
                       preferred_element_type=jnp.float32)
    start_of_tok = jnp.sum(starts_f * oh_f, axis=1, keepdims=True)

    slot_ref[...] = (start_of_tok + rank).astype(jnp.int32)
    starts_ref[...] = starts_f.astype(jnp.int32)
    nblocks_ref[...] = (counts + (_TILE - 1)) // _TILE


def _dispatch_sc(hs, w128, idx2d, npad):
    n, d = hs.shape
    dw = w128.shape[1]
    info = plsc.get_sparse_core_info()
    chunk = idx2d.shape[1]
    mesh = plsc.VectorSubcoreMesh(core_axis_name="c", subcore_axis_name="s")

    @functools.partial(
        pl.kernel,
        mesh=mesh,
        out_type=(
            jax.ShapeDtypeStruct((npad, d), jnp.float32),
            jax.ShapeDtypeStruct((npad, dw), jnp.float32),
        ),
        scratch_types=[
            pltpu.VMEM((chunk,), jnp.int32),
            pltpu.VMEM((chunk, d), jnp.float32),
            pltpu.VMEM((chunk, dw), jnp.float32),
            pltpu.SemaphoreType.DMA,
        ],
    )
    def scat(hs_hbm, w_hbm, idx_hbm, xs_hbm, ws_hbm, idx_v, rows_v, w_v,
             sem):
        wid = (jax.lax.axis_index("s") * info.num_cores
               + jax.lax.axis_index("c"))
        base = wid * chunk
        pltpu.sync_copy(idx_hbm.at[wid], idx_v)
        pltpu.sync_copy(hs_hbm.at[pl.ds(base, chunk)], rows_v)
        pltpu.async_copy(rows_v, xs_hbm.at[idx_v], sem).wait()
        pltpu.sync_copy(w_hbm.at[pl.ds(base, chunk)], w_v)
        pltpu.async_copy(w_v, ws_hbm.at[idx_v], sem).wait()

    return scat(hs, w128, idx2d)


def _unsort_sc(ys, idx2d, n):
    npad, d = ys.shape
    info = plsc.get_sparse_core_info()
    chunk = idx2d.shape[1]
    mesh = plsc.VectorSubcoreMesh(core_axis_name="c", subcore_axis_name="s")

    @functools.partial(
        pl.kernel,
        mesh=mesh,
        out_type=jax.ShapeDtypeStruct((n, d), jnp.float32),
        scratch_types=[
            pltpu.VMEM((chunk,), jnp.int32),
            pltpu.VMEM((chunk, d), jnp.float32),
            pltpu.SemaphoreType.DMA,
        ],
    )
    def gath(ys_hbm, idx_hbm, out_hbm, idx_v, rows_v, sem):
        wid = (jax.lax.axis_index("s") * info.num_cores
               + jax.lax.axis_index("c"))
        base = wid * chunk
        pltpu.sync_copy(idx_hbm.at[wid], idx_v)
        pltpu.async_copy(ys_hbm.at[idx_v], rows_v, sem).wait()
        pltpu.sync_copy(rows_v, out_hbm.at[pl.ds(base, chunk)])

    return gath(ys, idx2d)


def _moe_body(starts_ref, nblocks_ref, xs_ref, ws_ref, wg_ref, wu_ref,
              wd_ref, out_ref):
    e = pl.program_id(0)
    start = starts_ref[e]
    nb = nblocks_ref[e]
    wg = wg_ref[0].astype(jnp.bfloat16)
    wu = wu_ref[0].astype(jnp.bfloat16)
    wd = wd_ref[0].astype(jnp.bfloat16)

    def tile(k, carry):
        offs = pl.multiple_of(start + k * _TILE, 8)
        x = xs_ref[pl.ds(offs, _TILE), :].astype(jnp.bfloat16)
        g = jnp.dot(x, wg, preferred_element_type=jnp.float32)
        u = jnp.dot(x, wu, preferred_element_type=jnp.float32)
        a = ((g * jax.nn.sigmoid(g)) * u).astype(jnp.bfloat16)
        o = jnp.dot(a, wd, preferred_element_type=jnp.float32)
        w = ws_ref[pl.ds(offs, _TILE), 0:1]
        out_ref[pl.ds(offs, _TILE), :] = o * w
        return carry

    jax.lax.fori_loop(0, nb, tile, 0)


def kernel(hidden_states, top_k_indices, top_k_weights, Wg, Wu, Wd):
    N, D = hidden_states.shape
    E, _, H = Wg.shape
    K = top_k_indices.shape[1]
    NK = N * K

    npad = NK + 8 * E + 4 * _TILE
    npad = ((npad + 255) // 256) * 256

    eid = top_k_indices.reshape(NK, 1).astype(jnp.int32)
    wts = top_k_weights.reshape(NK, 1).astype(jnp.float32)

    slot, w128, starts, nblocks = pl.pallas_call(
        _route_body,
        out_shape=(
            jax.ShapeDtypeStruct((NK, 1), jnp.int32),
            jax.ShapeDtypeStruct((NK, 128), jnp.float32),
            jax.ShapeDtypeStruct((1, E), jnp.int32),
            jax.ShapeDtypeStruct((1, E), jnp.int32),
        ),
    )(eid, wts)
    nw = (plsc.get_sparse_core_info().num_cores
          * plsc.get_sparse_core_info().num_subcores)
    idx2d = slot.reshape(nw, NK // nw) if NK % nw == 0 else None

    if K > 1:
        hs = hidden_states[
            jnp.repeat(jnp.arange(N, dtype=jnp.int32), K)]
    else:
        hs = hidden_states
    xs, ws = _dispatch_sc(hs, w128, idx2d, npad)

    ys = pl.pallas_call(
        _moe_body,
        grid_spec=pltpu.PrefetchScalarGridSpec(
            num_scalar_prefetch=2,
            grid=(E,),
            in_specs=[
                pl.BlockSpec((npad, D), lambda e, s, nb: (0, 0)),
                pl.BlockSpec((npad, 128), lambda e, s, nb: (0, 0)),
                pl.BlockSpec((1, D, H), lambda e, s, nb: (e, 0, 0)),
                pl.BlockSpec((1, D, H), lambda e, s, nb: (e, 0, 0)),
                pl.BlockSpec((1, H, D), lambda e, s, nb: (e, 0, 0)),
            ],
            out_specs=pl.BlockSpec((npad, D), lambda e, s, nb: (0, 0)),
        ),
        out_shape=jax.ShapeDtypeStruct((npad, D), jnp.float32),
        compiler_params=pltpu.CompilerParams(
            dimension_semantics=("arbitrary",)),
    )(starts.reshape(E), nblocks.reshape(E), xs, ws, Wg, Wu, Wd)

    if K == 1:
        return _unsort_sc(ys, idx2d, N)
    slot2 = slot.reshape(N, K)
    out = ys[slot2[:, 0]]
    for k in range(1, K):
        out = out + ys[slot2[:, k]]
    return out


# speedup vs baseline: 1.4294x; 1.0036x over previous
import functools

import jax
import jax.numpy as jnp
from jax.experimental import pallas as pl
from jax.experimental.pallas import tpu as pltpu
from jax.experimental.pallas import tpu_sc as plsc

_TILE = 64


def _route_body(eid_ref, wts_ref, slot_ref, w128_ref, starts_ref,
                nblocks_ref):
    nk = eid_ref.shape[0]
    num_e = starts_ref.shape[1]
    w128_ref[...] = jnp.broadcast_to(wts_ref[...], (nk, 128))

    eid = eid_ref[...]
    lanes = jax.lax.broadcasted_iota(jnp.int32, (nk, num_e), 1)
    oh = eid == lanes
    oh_bf = oh.astype(jnp.bfloat16)
    oh_f = oh.astype(jnp.float32)

    row = jax.lax.broadcasted_iota(jnp.int32, (nk, nk), 0)
    col = jax.lax.broadcasted_iota(jnp.int32, (nk, nk), 1)
    lower = (col < row).astype(jnp.bfloat16)
    before = jnp.dot(lower, oh_bf, preferred_element_type=jnp.float32)
    rank = jnp.sum(before * oh_f, axis=1, keepdims=True)

    counts = jnp.sum(oh_f, axis=0, keepdims=True).astype(jnp.int32)
    aligned = ((counts + 7) // 8) * 8
    erow = jax.lax.broadcasted_iota(jnp.int32, (num_e, num_e), 0)
    ecol = jax.lax.broadcasted_iota(jnp.int32, (num_e, num_e), 1)
    tri = (erow < ecol).astype(jnp.bfloat16)
    starts_f = jnp.dot(aligned.astype(jnp.bfloat16), tri,
                       preferred_element_type=jnp.float32)
    start_of_tok = jnp.sum(starts_f * oh_f, axis=1, keepdims=True)

    slot_ref[...] = (start_of_tok + rank).astype(jnp.int32)
    starts_ref[...] = starts_f.astype(jnp.int32)
    nblocks_ref[...] = (counts + (_TILE - 1)) // _TILE


def _dispatch_sc(hs, w128, idx2d, npad):
    n, d = hs.shape
    dw = w128.shape[1]
    info = plsc.get_sparse_core_info()
    chunk = idx2d.shape[1]
    mesh = plsc.VectorSubcoreMesh(core_axis_name="c", subcore_axis_name="s")

    @functools.partial(
        pl.kernel,
        mesh=mesh,
        out_type=(
            jax.ShapeDtypeStruct((npad, d), jnp.float32),
            jax.ShapeDtypeStruct((npad, dw), jnp.float32),
        ),
        scratch_types=[
            pltpu.VMEM((chunk,), jnp.int32),
            pltpu.VMEM((chunk, d), jnp.float32),
            pltpu.VMEM((chunk, dw), jnp.float32),
            pltpu.SemaphoreType.DMA,
            pltpu.SemaphoreType.DMA,
            pltpu.SemaphoreType.DMA,
            pltpu.SemaphoreType.DMA,
        ],
    )
    def scat(hs_hbm, w_hbm, idx_hbm, xs_hbm, ws_hbm, idx_v, rows_v, w_v,
             sem_a, sem_b, sem_c, sem_d):
        wid = (jax.lax.axis_index("s") * info.num_cores
               + jax.lax.axis_index("c"))
        base = wid * chunk
        pltpu.sync_copy(idx_hbm.at[wid], idx_v)
        lda = pltpu.async_copy(hs_hbm.at[pl.ds(base, chunk)], rows_v, sem_a)
        ldb = pltpu.async_copy(w_hbm.at[pl.ds(base, chunk)], w_v, sem_b)
        lda.wait()
        sta = pltpu.async_copy(rows_v, xs_hbm.at[idx_v], sem_c)
        ldb.wait()
        stb = pltpu.async_copy(w_v, ws_hbm.at[idx_v], sem_d)
        sta.wait()
        stb.wait()

    return scat(hs, w128, idx2d)


def _unsort_sc(ys, idx2d, n):
    npad, d = ys.shape
    info = plsc.get_sparse_core_info()
    chunk = idx2d.shape[1]
    mesh = plsc.VectorSubcoreMesh(core_axis_name="c", subcore_axis_name="s")

    @functools.partial(
        pl.kernel,
        mesh=mesh,
        out_type=jax.ShapeDtypeStruct((n, d), jnp.float32),
        scratch_types=[
            pltpu.VMEM((chunk,), jnp.int32),
            pltpu.VMEM((chunk, d), jnp.float32),
            pltpu.SemaphoreType.DMA,
        ],
    )
    def gath(ys_hbm, idx_hbm, out_hbm, idx_v, rows_v, sem):
        wid = (jax.lax.axis_index("s") * info.num_cores
               + jax.lax.axis_index("c"))
        base = wid * chunk
        pltpu.sync_copy(idx_hbm.at[wid], idx_v)
        pltpu.async_copy(ys_hbm.at[idx_v], rows_v, sem).wait()
        pltpu.sync_copy(rows_v, out_hbm.at[pl.ds(base, chunk)])

    return gath(ys, idx2d)


def _moe_body(starts_ref, nblocks_ref, xs_ref, ws_ref, wg_ref, wu_ref,
              wd_ref, out_ref):
    e = pl.program_id(0)
    start = starts_ref[e]
    nb = nblocks_ref[e]
    wg = wg_ref[0].astype(jnp.bfloat16)
    wu = wu_ref[0].astype(jnp.bfloat16)
    wd = wd_ref[0].astype(jnp.bfloat16)

    def tile(k, carry):
        offs = pl.multiple_of(start + k * _TILE, 8)
        x = xs_ref[pl.ds(offs, _TILE), :].astype(jnp.bfloat16)
        g = jnp.dot(x, wg, preferred_element_type=jnp.float32)
        u = jnp.dot(x, wu, preferred_element_type=jnp.float32)
        a = ((g * jax.nn.sigmoid(g)) * u).astype(jnp.bfloat16)
        o = jnp.dot(a, wd, preferred_element_type=jnp.float32)
        w = ws_ref[pl.ds(offs, _TILE), 0:1]
        out_ref[pl.ds(offs, _TILE), :] = o * w
        return carry

    jax.lax.fori_loop(0, nb, tile, 0)


def kernel(hidden_states, top_k_indices, top_k_weights, Wg, Wu, Wd):
    N, D = hidden_states.shape
    E, _, H = Wg.shape
    K = top_k_indices.shape[1]
    NK = N * K

    npad = NK + 8 * E + 4 * _TILE
    npad = ((npad + 255) // 256) * 256

    eid = top_k_indices.reshape(NK, 1).astype(jnp.int32)
    wts = top_k_weights.reshape(NK, 1).astype(jnp.float32)

    slot, w128, starts, nblocks = pl.pallas_call(
        _route_body,
        out_shape=(
            jax.ShapeDtypeStruct((NK, 1), jnp.int32),
            jax.ShapeDtypeStruct((NK, 128), jnp.float32),
            jax.ShapeDtypeStruct((1, E), jnp.int32),
            jax.ShapeDtypeStruct((1, E), jnp.int32),
        ),
    )(eid, wts)
    nw = (plsc.get_sparse_core_info().num_cores
          * plsc.get_sparse_core_info().num_subcores)
    idx2d = slot.reshape(nw, NK // nw) if NK % nw == 0 else None

    if K > 1:
        hs = hidden_states[
            jnp.repeat(jnp.arange(N, dtype=jnp.int32), K)]
    else:
        hs = hidden_states
    xs, ws = _dispatch_sc(hs, w128, idx2d, npad)

    ys = pl.pallas_call(
        _moe_body,
        grid_spec=pltpu.PrefetchScalarGridSpec(
            num_scalar_prefetch=2,
            grid=(E,),
            in_specs=[
                pl.BlockSpec((npad, D), lambda e, s, nb: (0, 0)),
                pl.BlockSpec((npad, 128), lambda e, s, nb: (0, 0)),
                pl.BlockSpec((1, D, H), lambda e, s, nb: (e, 0, 0)),
                pl.BlockSpec((1, D, H), lambda e, s, nb: (e, 0, 0)),
                pl.BlockSpec((1, H, D), lambda e, s, nb: (e, 0, 0)),
            ],
            out_specs=pl.BlockSpec((npad, D), lambda e, s, nb: (0, 0)),
        ),
        out_shape=jax.ShapeDtypeStruct((npad, D), jnp.float32),
        compiler_params=pltpu.CompilerParams(
            dimension_semantics=("arbitrary",)),
    )(starts.reshape(E), nblocks.reshape(E), xs, ws, Wg, Wu, Wd)

    if K == 1:
        return _unsort_sc(ys, idx2d, N)
    slot2 = slot.reshape(N, K)
    out = ys[slot2[:, 0]]
    for k in range(1, K):
        out = out + ys[slot2[:, k]]
    return out
